# baseline (speedup 1.0000x reference)
import jax
import jax.numpy as jnp
from jax import lax
from jax.experimental import pallas as pl
from jax.experimental.pallas import tpu as pltpu
from jax.experimental.pallas import tpu_sc as plsc

D = 128
NC, NS = 2, 16
NW = NC * NS
NBUF = 4
ROWS = 128


def _embed_kernel(x_hbm, tok_hbm, pos_hbm, out_hbm,
                  idx_v, b0, b1, b2, b3, pos_s,
                  isem, psems, gsems, ssems, stsem):
    batch, seq_len = x_hbm.shape
    s_per_w = seq_len // NW
    c_per_w = s_per_w // D
    n_tasks = batch * c_per_w

    sid = lax.axis_index("s")
    wid = sid * NC + lax.axis_index("c")
    s0 = wid * s_per_w
    p0 = sid * s_per_w
    bufs = [b0, b1, b2, b3]

    def task_src(t):
        b, h = divmod(t, c_per_w)
        return b, s0 + h * ROWS

    hidx = pltpu.async_copy(
        x_hbm.at[pl.ds(0, batch), pl.ds(s0, s_per_w)],
        idx_v, isem)

    hstage = pltpu.async_copy(pos_hbm.at[pl.ds(s0, s_per_w)],
                              pos_s.at[pl.ds(p0, s_per_w)], stsem)

    def prefill(t):
        _, h = divmod(t, c_per_w)
        if t < NBUF:
            return pltpu.async_copy(pos_hbm.at[pl.ds(s0 + h * ROWS, ROWS)],
                                    bufs[t % NBUF], psems.at[t % NBUF])
        return pltpu.async_copy(pos_s.at[pl.ds(p0 + h * ROWS, ROWS)],
                                bufs[t % NBUF], psems.at[t % NBUF])

    hpre = {t: prefill(t) for t in range(NBUF)}
    hidx.wait()

    hg, hst = {}, {}

    def store(t):
        b, s = task_src(t)
        return pltpu.async_copy(bufs[t % NBUF],
                                out_hbm.at[b, pl.ds(s, ROWS)],
                                ssems.at[t % NBUF])

    for t in range(n_tasks):
        B = t % NBUF
        hpre[t].wait()
        b, h = divmod(t, c_per_w)
        hg[t] = pltpu.async_copy(tok_hbm.at[idx_v.at[b, pl.ds(h * ROWS, ROWS)]],
                                 bufs[B], gsems.at[B], add=True)
        if t >= 1:
            hg[t - 1].wait()
            hst[t - 1] = store(t - 1)
        if t == 2:
            hstage.wait()
        if t >= 2 and t + 2 < n_tasks:
            hst[t - 2].wait()
            hpre[t + 2] = prefill(t + 2)
    t = n_tasks - 1
    hg[t].wait()
    hst[t] = store(t)
    for t in range(n_tasks - 2, n_tasks):
        hst[t].wait()


def kernel(x, token_table, pos_table):
    batch, seq_len = x.shape

    mesh = plsc.VectorSubcoreMesh(core_axis_name="c", subcore_axis_name="s")
    run = pl.kernel(
        _embed_kernel,
        mesh=mesh,
        out_type=jax.ShapeDtypeStruct((batch, seq_len, D), jnp.float32),
        scratch_types=[
            pltpu.VMEM((batch, seq_len // NW), jnp.int32),
            pltpu.VMEM((ROWS, D), jnp.float32),
            pltpu.VMEM((ROWS, D), jnp.float32),
            pltpu.VMEM((ROWS, D), jnp.float32),
            pltpu.VMEM((ROWS, D), jnp.float32),
            pltpu.VMEM_SHARED((NS * (seq_len // NW), D),
                              jnp.float32),
            pltpu.SemaphoreType.DMA,
            pltpu.SemaphoreType.DMA((NBUF,)),
            pltpu.SemaphoreType.DMA((NBUF,)),
            pltpu.SemaphoreType.DMA((NBUF,)),
            pltpu.SemaphoreType.DMA,
        ],
    )
    return run(x.astype(jnp.int32), token_table, pos_table)


# speedup vs baseline: 1.0181x; 1.0181x over previous
import jax
import jax.numpy as jnp
from jax import lax
from jax.experimental import pallas as pl
from jax.experimental.pallas import tpu as pltpu
from jax.experimental.pallas import tpu_sc as plsc

D = 128
NC, NS = 2, 16
NW = NC * NS
NBUF = 4
ROWS = 128


def _embed_kernel(x_hbm, tok_hbm, pos_hbm, out_hbm,
                  idx_v, b0, b1, b2, b3, pos_s,
                  isem, psems, gsems, ssems, stsem):
    batch, seq_len = x_hbm.shape
    s_per_w = seq_len // NW
    c_per_w = s_per_w // D
    n_tasks = batch * c_per_w

    sid = lax.axis_index("s")
    wid = sid * NC + lax.axis_index("c")
    s0 = wid * s_per_w
    p0 = sid * s_per_w
    bufs = [b0, b1, b2, b3]

    def task_src(t):
        b, h = divmod(t, c_per_w)
        return b, s0 + h * ROWS

    hidx = pltpu.async_copy(
        x_hbm.at[pl.ds(0, batch), pl.ds(s0, s_per_w)],
        idx_v, isem)

    pltpu.async_copy(pos_hbm.at[pl.ds(s0, s_per_w)],
                     pos_s.at[pl.ds(p0, s_per_w)], stsem).wait()

    def prefill(t):
        _, h = divmod(t, c_per_w)
        return pltpu.async_copy(pos_s.at[pl.ds(p0 + h * ROWS, ROWS)],
                                bufs[t % NBUF], psems.at[t % NBUF])

    hpre = {t: prefill(t) for t in range(NBUF)}
    hidx.wait()

    hg, hst = {}, {}

    def store(t):
        b, s = task_src(t)
        return pltpu.async_copy(bufs[t % NBUF],
                                out_hbm.at[b, pl.ds(s, ROWS)],
                                ssems.at[t % NBUF])

    for t in range(n_tasks):
        B = t % NBUF
        hpre[t].wait()
        b, h = divmod(t, c_per_w)
        hg[t] = pltpu.async_copy(tok_hbm.at[idx_v.at[b, pl.ds(h * ROWS, ROWS)]],
                                 bufs[B], gsems.at[B], add=True)
        if t >= 1:
            hg[t - 1].wait()
            hst[t - 1] = store(t - 1)
        if t >= 2 and t + 2 < n_tasks:
            hst[t - 2].wait()
            hpre[t + 2] = prefill(t + 2)
    t = n_tasks - 1
    hg[t].wait()
    hst[t] = store(t)
    for t in range(n_tasks - 2, n_tasks):
        hst[t].wait()


def kernel(x, token_table, pos_table):
    batch, seq_len = x.shape

    mesh = plsc.VectorSubcoreMesh(core_axis_name="c", subcore_axis_name="s")
    run = pl.kernel(
        _embed_kernel,
        mesh=mesh,
        out_type=jax.ShapeDtypeStruct((batch, seq_len, D), jnp.float32),
        scratch_types=[
            pltpu.VMEM((batch, seq_len // NW), jnp.int32),
            pltpu.VMEM((ROWS, D), jnp.float32),
            pltpu.VMEM((ROWS, D), jnp.float32),
            pltpu.VMEM((ROWS, D), jnp.float32),
            pltpu.VMEM((ROWS, D), jnp.float32),
            pltpu.VMEM_SHARED((NS * (seq_len // NW), D),
                              jnp.float32),
            pltpu.SemaphoreType.DMA,
            pltpu.SemaphoreType.DMA((NBUF,)),
            pltpu.SemaphoreType.DMA((NBUF,)),
            pltpu.SemaphoreType.DMA((NBUF,)),
            pltpu.SemaphoreType.DMA,
        ],
    )
    return run(x.astype(jnp.int32), token_table, pos_table)
